# baseline (speedup 1.0000x reference)
import functools

import jax
import jax.numpy as jnp
from jax import lax
from jax.experimental import pallas as pl
from jax.experimental.pallas import tpu as pltpu
from jax.experimental.pallas import tpu_sc as plsc

_NC = 2
_NS = 16
_NW = _NC * _NS


def _sc_degree(dst_r, zeros_n16):
    nw, nch, c = dst_r.shape
    n = zeros_n16.shape[0]
    rps = (n // _NS) // 8 * 8
    tail0, tail = rps * _NS, n - rps * _NS
    mesh = plsc.VectorSubcoreMesh(core_axis_name="c", subcore_axis_name="s")

    @functools.partial(
        pl.kernel,
        mesh=mesh,
        out_type=jax.ShapeDtypeStruct((_NC, n, 16), jnp.float32),
        scratch_types=[
            pltpu.VMEM((nch, c), jnp.int32),
            pltpu.VMEM((c, 16), jnp.float32),
            pltpu.VMEM_SHARED((n, 16), jnp.float32),
        ],
    )
    def deg_kernel(dst_hbm, z_hbm, out_hbm, dstv, ones, acc):
        ci = lax.axis_index("c")
        si = lax.axis_index("s")
        wid = ci * _NS + si
        r0 = pl.multiple_of(si * rps, 8)
        pltpu.sync_copy(z_hbm.at[pl.ds(r0, rps)], acc.at[pl.ds(r0, rps)])
        if tail:
            @pl.when(si == _NS - 1)
            def _():
                pltpu.sync_copy(z_hbm.at[pl.ds(tail0, tail)],
                                acc.at[pl.ds(tail0, tail)])
        pltpu.sync_copy(dst_hbm.at[wid], dstv)

        @pl.loop(0, c)
        def _(i):
            ones[i] = jnp.ones((16,), jnp.float32)

        plsc.subcore_barrier()

        @pl.loop(0, nch)
        def _(g):
            pltpu.sync_copy(ones, acc.at[dstv.at[g]], add=True)

        plsc.subcore_barrier()
        pltpu.sync_copy(acc.at[pl.ds(r0, rps)],
                        out_hbm.at[ci].at[pl.ds(r0, rps)])
        if tail:
            @pl.when(si == _NS - 1)
            def _():
                pltpu.sync_copy(acc.at[pl.ds(tail0, tail)],
                                out_hbm.at[ci].at[pl.ds(tail0, tail)])

    return deg_kernel(dst_r, zeros_n16)


_NBUF = 2


@functools.lru_cache(maxsize=None)
def _make_sc_scatter_add(n, dh, nch, c):
    rps = (n // _NS) // 8 * 8
    tail0, tail = rps * _NS, n - rps * _NS
    assert nch % _NBUF == 0 and nch >= 2 * _NBUF
    mesh = plsc.VectorSubcoreMesh(core_axis_name="c", subcore_axis_name="s")

    @functools.partial(
        pl.kernel,
        mesh=mesh,
        out_type=jax.ShapeDtypeStruct((_NC, 2, n, dh), jnp.float32),
        compiler_params=pltpu.CompilerParams(use_tc_tiling_on_sc=False),
        scratch_types=[
            pltpu.VMEM((nch, c), jnp.int32),
            pltpu.VMEM((nch, c), jnp.int32),
            pltpu.VMEM((_NBUF, c, dh), jnp.float32),
            pltpu.VMEM_SHARED((n + 8, dh), jnp.float32),
            [pltpu.SemaphoreType.DMA] * _NBUF,
        ],
    )
    def scat_kernel(ylo_hbm, yhi_hbm, src_hbm, dst_hbm, z_hbm, out_hbm,
                    srcv, dstv, rows, acc, sems):
        ci = lax.axis_index("c")
        si = lax.axis_index("s")
        wid = ci * _NS + si
        r0 = pl.multiple_of(si * rps, 8)
        pltpu.sync_copy(src_hbm.at[wid], srcv)
        pltpu.sync_copy(dst_hbm.at[wid], dstv)

        for h, y_hbm in enumerate((ylo_hbm, yhi_hbm)):
            pltpu.sync_copy(z_hbm.at[pl.ds(r0, rps)], acc.at[pl.ds(r0, rps)])
            if tail:
                @pl.when(si == _NS - 1)
                def _():
                    pltpu.sync_copy(z_hbm.at[pl.ds(tail0, tail)],
                                    acc.at[pl.ds(tail0, tail)])
            plsc.subcore_barrier()

            for b in range(_NBUF):
                pltpu.async_copy(y_hbm.at[srcv.at[b]], rows.at[b], sems[b])

            @pl.loop(0, nch - _NBUF, step=_NBUF)
            def _(g):
                for b in range(_NBUF):
                    gg = g + b
                    pltpu.make_async_copy(
                        y_hbm.at[srcv.at[gg]], rows.at[b], sems[b]).wait()
                    pltpu.sync_copy(rows.at[b], acc.at[dstv.at[gg]], add=True)
                    pltpu.async_copy(
                        y_hbm.at[srcv.at[gg + _NBUF]], rows.at[b], sems[b])

            for b in range(_NBUF):
                gg = nch - _NBUF + b
                pltpu.make_async_copy(
                    y_hbm.at[srcv.at[gg]], rows.at[b], sems[b]).wait()
                pltpu.sync_copy(rows.at[b], acc.at[dstv.at[gg]], add=True)

            plsc.subcore_barrier()
            pltpu.sync_copy(acc.at[pl.ds(r0, rps)],
                            out_hbm.at[ci].at[h].at[pl.ds(r0, rps)])
            if tail:
                @pl.when(si == _NS - 1)
                def _():
                    pltpu.sync_copy(acc.at[pl.ds(tail0, tail)],
                                    out_hbm.at[ci].at[h].at[pl.ds(tail0, tail)])

    return scat_kernel


def _sc_scatter_add(y_lo, y_hi, src_r, dst_r, zeros_ndh):
    n, dh = y_lo.shape
    nw, nch, c = src_r.shape
    return _make_sc_scatter_add(n, dh, nch, c)(
        y_lo, y_hi, src_r, dst_r, zeros_ndh)


def _dinv_from_parts(dp):
    deg = dp[0, :, 0] + dp[1, :, 0] + 1.0
    return (1.0 / jnp.sqrt(deg))[:, None]


def _tc_stage1(x, w1, deg_p, block):
    n, d = x.shape
    dh = d // 2

    def body(x_ref, w_ref, dp_ref, lo_ref, hi_ref):
        dinv = _dinv_from_parts(dp_ref[...])
        xw = jnp.dot(x_ref[...], w_ref[...],
                     preferred_element_type=jnp.float32)
        y = xw * dinv
        lo_ref[...] = y[:, :dh]
        hi_ref[...] = y[:, dh:]

    return pl.pallas_call(
        body,
        grid=(n // block,),
        in_specs=[
            pl.BlockSpec((block, d), lambda i: (i, 0)),
            pl.BlockSpec((d, d), lambda i: (0, 0)),
            pl.BlockSpec((2, block, 16), lambda i: (0, i, 0)),
        ],
        out_specs=[
            pl.BlockSpec((block, dh), lambda i: (i, 0)),
            pl.BlockSpec((block, dh), lambda i: (i, 0)),
        ],
        out_shape=[
            jax.ShapeDtypeStruct((n, dh), jnp.float32),
            jax.ShapeDtypeStruct((n, dh), jnp.float32),
        ],
    )(x, w1, deg_p)


def _tc_stage2(s1, y1_lo, y1_hi, deg_p, w2, b1, block):
    n, dh = y1_lo.shape
    d = 2 * dh

    def body(s_ref, ylo_ref, yhi_ref, dp_ref, w_ref, b_ref,
             olo_ref, ohi_ref):
        dinv = _dinv_from_parts(dp_ref[...])
        sv = s_ref[...]
        b = b_ref[...]
        w = w_ref[...]
        lo = (sv[0, 0] + sv[1, 0] + ylo_ref[...]) * dinv + b[:, :dh]
        hi = (sv[0, 1] + sv[1, 1] + yhi_ref[...]) * dinv + b[:, dh:]
        h_lo = jnp.maximum(lo, 0.0)
        h_hi = jnp.maximum(hi, 0.0)
        hw = (jnp.dot(h_lo, w[:dh, :], preferred_element_type=jnp.float32)
              + jnp.dot(h_hi, w[dh:, :], preferred_element_type=jnp.float32))
        y2 = hw * dinv
        olo_ref[...] = y2[:, :dh]
        ohi_ref[...] = y2[:, dh:]

    return pl.pallas_call(
        body,
        grid=(n // block,),
        in_specs=[
            pl.BlockSpec((2, 2, block, dh), lambda i: (0, 0, i, 0)),
            pl.BlockSpec((block, dh), lambda i: (i, 0)),
            pl.BlockSpec((block, dh), lambda i: (i, 0)),
            pl.BlockSpec((2, block, 16), lambda i: (0, i, 0)),
            pl.BlockSpec((d, d), lambda i: (0, 0)),
            pl.BlockSpec((1, d), lambda i: (0, 0)),
        ],
        out_specs=[
            pl.BlockSpec((block, dh), lambda i: (i, 0)),
            pl.BlockSpec((block, dh), lambda i: (i, 0)),
        ],
        out_shape=[
            jax.ShapeDtypeStruct((n, dh), jnp.float32),
            jax.ShapeDtypeStruct((n, dh), jnp.float32),
        ],
    )(s1, y1_lo, y1_hi, deg_p, w2, b1)


def _tc_stage3(s2, y2_lo, y2_hi, deg_p, b2, wl_t, bl, block):
    n, dh = y2_lo.shape
    d = 2 * dh
    k = wl_t.shape[1]

    def body(s_ref, ylo_ref, yhi_ref, dp_ref, b2_ref, wl_ref, bl_ref,
             logp_ref, emb_ref):
        dinv = _dinv_from_parts(dp_ref[...])
        sv = s_ref[...]
        b2v = b2_ref[...]
        lo = (sv[0, 0] + sv[1, 0] + ylo_ref[...]) * dinv + b2v[:, :dh]
        hi = (sv[0, 1] + sv[1, 1] + yhi_ref[...]) * dinv + b2v[:, dh:]
        emb = jnp.concatenate([lo, hi], axis=1)
        logits = jnp.dot(emb, wl_ref[...],
                         preferred_element_type=jnp.float32) + bl_ref[...]
        m = jnp.max(logits, axis=1, keepdims=True)
        e = jnp.exp(logits - m)
        lse = m + jnp.log(jnp.sum(e, axis=1, keepdims=True))
        logp_ref[...] = logits - lse
        emb_ref[...] = emb

    return pl.pallas_call(
        body,
        grid=(n // block,),
        in_specs=[
            pl.BlockSpec((2, 2, block, dh), lambda i: (0, 0, i, 0)),
            pl.BlockSpec((block, dh), lambda i: (i, 0)),
            pl.BlockSpec((block, dh), lambda i: (i, 0)),
            pl.BlockSpec((2, block, 16), lambda i: (0, i, 0)),
            pl.BlockSpec((1, d), lambda i: (0, 0)),
            pl.BlockSpec((d, k), lambda i: (0, 0)),
            pl.BlockSpec((1, k), lambda i: (0, 0)),
        ],
        out_specs=[
            pl.BlockSpec((block, k), lambda i: (i, 0)),
            pl.BlockSpec((block, d), lambda i: (i, 0)),
        ],
        out_shape=[
            jax.ShapeDtypeStruct((n, k), jnp.float32),
            jax.ShapeDtypeStruct((n, d), jnp.float32),
        ],
    )(s2, y2_lo, y2_hi, deg_p, b2, wl_t, bl)


def kernel(x, edge_index, W1, b1, W2, b2, Wl, bl):
    n, d = x.shape
    dh = d // 2
    e = edge_index.shape[1]
    epw = e // _NW

    c_scat = 80
    c_deg = 80
    nch = -(-epw // (c_scat * _NBUF)) * _NBUF
    nch2 = epw // c_deg

    src = edge_index[0]
    dst = edge_index[1]
    pad = _NW * nch * c_scat - e
    src_p = jnp.concatenate([src, jnp.zeros((pad,), jnp.int32)])
    dst_p = jnp.concatenate([dst, jnp.full((pad,), n, jnp.int32)])
    src_r = src_p.reshape(_NW, nch, c_scat)
    dst_r = dst_p.reshape(_NW, nch, c_scat)
    dstd_r = dst.reshape(_NW, nch2, c_deg)
    zeros_n16 = jnp.zeros((n, 16), jnp.float32)
    zeros_ndh = jnp.zeros((n, dh), jnp.float32)

    block = 2000

    deg_p = _sc_degree(dstd_r, zeros_n16)
    y1_lo, y1_hi = _tc_stage1(x, W1, deg_p, block)
    s1 = _sc_scatter_add(y1_lo, y1_hi, src_r, dst_r, zeros_ndh)
    y2_lo, y2_hi = _tc_stage2(s1, y1_lo, y1_hi, deg_p, W2,
                              b1.reshape(1, d), block)
    s2 = _sc_scatter_add(y2_lo, y2_hi, src_r, dst_r, zeros_ndh)
    logp, x_emb = _tc_stage3(s2, y2_lo, y2_hi, deg_p,
                             b2.reshape(1, d), Wl.T,
                             bl.reshape(1, -1), block)
    return (logp, x_emb)


# speedup vs baseline: 1.1148x; 1.1148x over previous
import functools

import jax
import jax.numpy as jnp
from jax import lax
from jax.experimental import pallas as pl
from jax.experimental.pallas import tpu as pltpu
from jax.experimental.pallas import tpu_sc as plsc

_NC = 2
_NS = 16
_NW = _NC * _NS


def _sc_degree(dst_r, zeros_n16):
    nw, nch, c = dst_r.shape
    n = zeros_n16.shape[0]
    rps = (n // _NS) // 8 * 8
    tail0, tail = rps * _NS, n - rps * _NS
    mesh = plsc.VectorSubcoreMesh(core_axis_name="c", subcore_axis_name="s")

    @functools.partial(
        pl.kernel,
        mesh=mesh,
        out_type=jax.ShapeDtypeStruct((_NC, n, 16), jnp.float32),
        scratch_types=[
            pltpu.VMEM((nch, c), jnp.int32),
            pltpu.VMEM((c, 16), jnp.float32),
            pltpu.VMEM_SHARED((n, 16), jnp.float32),
        ],
    )
    def deg_kernel(dst_hbm, z_hbm, out_hbm, dstv, ones, acc):
        ci = lax.axis_index("c")
        si = lax.axis_index("s")
        wid = ci * _NS + si
        r0 = pl.multiple_of(si * rps, 8)
        pltpu.sync_copy(z_hbm.at[pl.ds(r0, rps)], acc.at[pl.ds(r0, rps)])
        if tail:
            @pl.when(si == _NS - 1)
            def _():
                pltpu.sync_copy(z_hbm.at[pl.ds(tail0, tail)],
                                acc.at[pl.ds(tail0, tail)])
        pltpu.sync_copy(dst_hbm.at[wid], dstv)

        @pl.loop(0, c)
        def _(i):
            ones[i] = jnp.ones((16,), jnp.float32)

        plsc.subcore_barrier()

        @pl.loop(0, nch)
        def _(g):
            pltpu.sync_copy(ones, acc.at[dstv.at[g]], add=True)

        plsc.subcore_barrier()
        pltpu.sync_copy(acc.at[pl.ds(r0, rps)],
                        out_hbm.at[ci].at[pl.ds(r0, rps)])
        if tail:
            @pl.when(si == _NS - 1)
            def _():
                pltpu.sync_copy(acc.at[pl.ds(tail0, tail)],
                                out_hbm.at[ci].at[pl.ds(tail0, tail)])

    return deg_kernel(dst_r, zeros_n16)


_NBUF = 2


@functools.lru_cache(maxsize=None)
def _make_sc_scatter_add(n, dh, nch, c):
    rps = (n // _NS) // 8 * 8
    tail0, tail = rps * _NS, n - rps * _NS
    assert nch % _NBUF == 0 and nch >= 2 * _NBUF
    mesh = plsc.VectorSubcoreMesh(core_axis_name="c", subcore_axis_name="s")

    @functools.partial(
        pl.kernel,
        mesh=mesh,
        out_type=jax.ShapeDtypeStruct((_NC, n, dh), jnp.float32),
        compiler_params=pltpu.CompilerParams(use_tc_tiling_on_sc=False),
        scratch_types=[
            pltpu.VMEM((nch, c), jnp.int32),
            pltpu.VMEM((nch, c), jnp.int32),
            pltpu.VMEM((_NBUF, c, dh), jnp.float32),
            pltpu.VMEM_SHARED((n + 8, dh), jnp.float32),
            [pltpu.SemaphoreType.DMA] * _NBUF,
        ],
    )
    def scat_kernel(ylo_hbm, yhi_hbm, src_hbm, dst_hbm, z_hbm, out_hbm,
                    srcv, dstv, rows, acc, sems):
        ci = lax.axis_index("c")
        si = lax.axis_index("s")
        r0 = pl.multiple_of(si * rps, 8)
        pltpu.sync_copy(src_hbm.at[si], srcv)
        pltpu.sync_copy(dst_hbm.at[si], dstv)
        pltpu.sync_copy(z_hbm.at[pl.ds(r0, rps)], acc.at[pl.ds(r0, rps)])
        if tail:
            @pl.when(si == _NS - 1)
            def _():
                pltpu.sync_copy(z_hbm.at[pl.ds(tail0, tail)],
                                acc.at[pl.ds(tail0, tail)])
        plsc.subcore_barrier()

        def run_half(y_hbm):
            for b in range(_NBUF):
                pltpu.async_copy(y_hbm.at[srcv.at[b]], rows.at[b], sems[b])

            @pl.loop(0, nch - _NBUF, step=_NBUF)
            def _(g):
                for b in range(_NBUF):
                    gg = g + b
                    pltpu.make_async_copy(
                        y_hbm.at[srcv.at[gg]], rows.at[b], sems[b]).wait()
                    pltpu.sync_copy(rows.at[b], acc.at[dstv.at[gg]], add=True)
                    pltpu.async_copy(
                        y_hbm.at[srcv.at[gg + _NBUF]], rows.at[b], sems[b])

            for b in range(_NBUF):
                gg = nch - _NBUF + b
                pltpu.make_async_copy(
                    y_hbm.at[srcv.at[gg]], rows.at[b], sems[b]).wait()
                pltpu.sync_copy(rows.at[b], acc.at[dstv.at[gg]], add=True)

        @pl.when(ci == 0)
        def _():
            run_half(ylo_hbm)

        @pl.when(ci == 1)
        def _():
            run_half(yhi_hbm)

        plsc.subcore_barrier()
        pltpu.sync_copy(acc.at[pl.ds(r0, rps)],
                        out_hbm.at[ci].at[pl.ds(r0, rps)])
        if tail:
            @pl.when(si == _NS - 1)
            def _():
                pltpu.sync_copy(acc.at[pl.ds(tail0, tail)],
                                out_hbm.at[ci].at[pl.ds(tail0, tail)])

    return scat_kernel


def _sc_scatter_add(y_lo, y_hi, src_r, dst_r, zeros_ndh):
    n, dh = y_lo.shape
    ns, nch, c = src_r.shape
    return _make_sc_scatter_add(n, dh, nch, c)(
        y_lo, y_hi, src_r, dst_r, zeros_ndh)


def _dinv_from_parts(dp):
    deg = dp[0, :, 0] + dp[1, :, 0] + 1.0
    return (1.0 / jnp.sqrt(deg))[:, None]


def _tc_stage1(x, w1, deg_p, block):
    n, d = x.shape
    dh = d // 2

    def body(x_ref, w_ref, dp_ref, lo_ref, hi_ref):
        dinv = _dinv_from_parts(dp_ref[...])
        xw = jnp.dot(x_ref[...], w_ref[...],
                     preferred_element_type=jnp.float32)
        y = xw * dinv
        lo_ref[...] = y[:, :dh]
        hi_ref[...] = y[:, dh:]

    return pl.pallas_call(
        body,
        grid=(n // block,),
        in_specs=[
            pl.BlockSpec((block, d), lambda i: (i, 0)),
            pl.BlockSpec((d, d), lambda i: (0, 0)),
            pl.BlockSpec((2, block, 16), lambda i: (0, i, 0)),
        ],
        out_specs=[
            pl.BlockSpec((block, dh), lambda i: (i, 0)),
            pl.BlockSpec((block, dh), lambda i: (i, 0)),
        ],
        out_shape=[
            jax.ShapeDtypeStruct((n, dh), jnp.float32),
            jax.ShapeDtypeStruct((n, dh), jnp.float32),
        ],
    )(x, w1, deg_p)


def _tc_stage2(s1, y1_lo, y1_hi, deg_p, w2, b1, block):
    n, dh = y1_lo.shape
    d = 2 * dh

    def body(s_ref, ylo_ref, yhi_ref, dp_ref, w_ref, b_ref,
             olo_ref, ohi_ref):
        dinv = _dinv_from_parts(dp_ref[...])
        sv = s_ref[...]
        b = b_ref[...]
        w = w_ref[...]
        lo = (sv[0] + ylo_ref[...]) * dinv + b[:, :dh]
        hi = (sv[1] + yhi_ref[...]) * dinv + b[:, dh:]
        h_lo = jnp.maximum(lo, 0.0)
        h_hi = jnp.maximum(hi, 0.0)
        hw = (jnp.dot(h_lo, w[:dh, :], preferred_element_type=jnp.float32)
              + jnp.dot(h_hi, w[dh:, :], preferred_element_type=jnp.float32))
        y2 = hw * dinv
        olo_ref[...] = y2[:, :dh]
        ohi_ref[...] = y2[:, dh:]

    return pl.pallas_call(
        body,
        grid=(n // block,),
        in_specs=[
            pl.BlockSpec((2, block, dh), lambda i: (0, i, 0)),
            pl.BlockSpec((block, dh), lambda i: (i, 0)),
            pl.BlockSpec((block, dh), lambda i: (i, 0)),
            pl.BlockSpec((2, block, 16), lambda i: (0, i, 0)),
            pl.BlockSpec((d, d), lambda i: (0, 0)),
            pl.BlockSpec((1, d), lambda i: (0, 0)),
        ],
        out_specs=[
            pl.BlockSpec((block, dh), lambda i: (i, 0)),
            pl.BlockSpec((block, dh), lambda i: (i, 0)),
        ],
        out_shape=[
            jax.ShapeDtypeStruct((n, dh), jnp.float32),
            jax.ShapeDtypeStruct((n, dh), jnp.float32),
        ],
    )(s1, y1_lo, y1_hi, deg_p, w2, b1)


def _tc_stage3(s2, y2_lo, y2_hi, deg_p, b2, wl_t, bl, block):
    n, dh = y2_lo.shape
    d = 2 * dh
    k = wl_t.shape[1]

    def body(s_ref, ylo_ref, yhi_ref, dp_ref, b2_ref, wl_ref, bl_ref,
             logp_ref, emb_ref):
        dinv = _dinv_from_parts(dp_ref[...])
        sv = s_ref[...]
        b2v = b2_ref[...]
        lo = (sv[0] + ylo_ref[...]) * dinv + b2v[:, :dh]
        hi = (sv[1] + yhi_ref[...]) * dinv + b2v[:, dh:]
        emb = jnp.concatenate([lo, hi], axis=1)
        logits = jnp.dot(emb, wl_ref[...],
                         preferred_element_type=jnp.float32) + bl_ref[...]
        m = jnp.max(logits, axis=1, keepdims=True)
        e = jnp.exp(logits - m)
        lse = m + jnp.log(jnp.sum(e, axis=1, keepdims=True))
        logp_ref[...] = logits - lse
        emb_ref[...] = emb

    return pl.pallas_call(
        body,
        grid=(n // block,),
        in_specs=[
            pl.BlockSpec((2, block, dh), lambda i: (0, i, 0)),
            pl.BlockSpec((block, dh), lambda i: (i, 0)),
            pl.BlockSpec((block, dh), lambda i: (i, 0)),
            pl.BlockSpec((2, block, 16), lambda i: (0, i, 0)),
            pl.BlockSpec((1, d), lambda i: (0, 0)),
            pl.BlockSpec((d, k), lambda i: (0, 0)),
            pl.BlockSpec((1, k), lambda i: (0, 0)),
        ],
        out_specs=[
            pl.BlockSpec((block, k), lambda i: (i, 0)),
            pl.BlockSpec((block, d), lambda i: (i, 0)),
        ],
        out_shape=[
            jax.ShapeDtypeStruct((n, k), jnp.float32),
            jax.ShapeDtypeStruct((n, d), jnp.float32),
        ],
    )(s2, y2_lo, y2_hi, deg_p, b2, wl_t, bl)


def kernel(x, edge_index, W1, b1, W2, b2, Wl, bl):
    n, d = x.shape
    dh = d // 2
    e = edge_index.shape[1]
    eps = e // _NS

    c_scat = 40
    c_deg = 80
    nch = -(-eps // (c_scat * _NBUF)) * _NBUF
    nch2 = (e // _NW) // c_deg

    src = edge_index[0]
    dst = edge_index[1]
    pad = _NS * nch * c_scat - e
    src_p = jnp.concatenate([src, jnp.zeros((pad,), jnp.int32)])
    dst_p = jnp.concatenate([dst, jnp.full((pad,), n, jnp.int32)])
    src_r = src_p.reshape(_NS, nch, c_scat)
    dst_r = dst_p.reshape(_NS, nch, c_scat)
    dstd_r = dst.reshape(_NW, nch2, c_deg)
    zeros_n16 = jnp.zeros((n, 16), jnp.float32)
    zeros_ndh = jnp.zeros((n, dh), jnp.float32)

    block = 2000

    deg_p = _sc_degree(dstd_r, zeros_n16)
    y1_lo, y1_hi = _tc_stage1(x, W1, deg_p, block)
    s1 = _sc_scatter_add(y1_lo, y1_hi, src_r, dst_r, zeros_ndh)
    y2_lo, y2_hi = _tc_stage2(s1, y1_lo, y1_hi, deg_p, W2,
                              b1.reshape(1, d), block)
    s2 = _sc_scatter_add(y2_lo, y2_hi, src_r, dst_r, zeros_ndh)
    logp, x_emb = _tc_stage3(s2, y2_lo, y2_hi, deg_p,
                             b2.reshape(1, d), Wl.T,
                             bl.reshape(1, -1), block)
    return (logp, x_emb)
